# Initial kernel scaffold; baseline (speedup 1.0000x reference)
import jax
import jax.numpy as jnp
from jax.experimental import pallas as pl


def kernel(x, edge_index, edge_weight, W_enc, b_enc, Wx1, bx1, Wx2, bx2, Wh1, bh1, Wh2, bh2):
    raise NotImplementedError("write your pallas kernel here")


# speedup vs baseline: 7.9059x; 7.9059x over previous
import functools

import jax
import jax.numpy as jnp
from jax import lax
from jax.experimental import pallas as pl
from jax.experimental.pallas import tpu as pltpu
from jax.experimental.pallas import tpu_sc as plsc

N = 10000
NPAD = 10240
E = 320000
F_DIM = 128
H_DIM = 64
DEC_H = 128
AUG = 144

NC = 2
NS = 16
CHUNK = 80


def _seg_sum_sc(d):
    edges_per_core = E // NC
    edges_per_sub = E // (NC * NS)
    n_chunks = edges_per_sub // CHUNK
    rows_per_sub = NPAD // NS

    mesh = plsc.VectorSubcoreMesh(
        core_axis_name="c", subcore_axis_name="s",
        num_cores=NC, num_subcores=NS)

    @functools.partial(
        pl.kernel,
        out_type=jax.ShapeDtypeStruct((NC, NPAD, d), jnp.float32),
        mesh=mesh,
        scratch_types=[
            pltpu.VMEM((CHUNK,), jnp.int32),
            pltpu.VMEM((CHUNK,), jnp.int32),
            pltpu.VMEM((CHUNK, d), jnp.float32),
            pltpu.VMEM_SHARED((NPAD, d), jnp.float32),
            pltpu.SemaphoreType.DMA,
        ],
        compiler_params=pltpu.CompilerParams(use_tc_tiling_on_sc=False),
    )
    def seg_sum(table_hbm, src_hbm, dst_hbm, zeros_hbm, out_hbm,
                src_v, dst_v, rows_v, acc_sh, sem):
        c = lax.axis_index("c")
        s = lax.axis_index("s")
        row0 = s * rows_per_sub
        pltpu.sync_copy(zeros_hbm.at[pl.ds(row0, rows_per_sub)],
                        acc_sh.at[pl.ds(row0, rows_per_sub)])
        plsc.subcore_barrier()

        base = c * edges_per_core + s * edges_per_sub

        def body(i, _):
            off = base + i * CHUNK
            pltpu.sync_copy(src_hbm.at[pl.ds(off, CHUNK)], src_v)
            pltpu.sync_copy(dst_hbm.at[pl.ds(off, CHUNK)], dst_v)
            pltpu.async_copy(table_hbm.at[src_v], rows_v, sem).wait()
            pltpu.sync_copy(rows_v, acc_sh.at[dst_v], add=True)
            return 0

        lax.fori_loop(0, n_chunks, body, 0)
        plsc.subcore_barrier()
        pltpu.sync_copy(acc_sh.at[pl.ds(row0, rows_per_sub)],
                        out_hbm.at[c, pl.ds(row0, rows_per_sub)])

    return seg_sum


_seg_sum_aug = _seg_sum_sc(AUG)
_seg_sum_h = _seg_sum_sc(H_DIM)


def _enc_body(p_ref, x_ref, w_ref, b_ref, m_ref, h_ref, ic_ref):
    ssum = p_ref[0] + p_ref[1]
    inv = 1.0 / (ssum[:, 128:129] + 1.0)
    m = (ssum[:, :F_DIM] + x_ref[...]) * inv
    m_ref[...] = m
    h_ref[...] = jnp.maximum(
        jnp.dot(m, w_ref[...], preferred_element_type=jnp.float32,
                precision=lax.Precision.HIGHEST) + b_ref[...], 0.0)
    ic_ref[...] = inv


def _dec_body(s2_ref, h_ref, x_ref, m_ref, ic_ref,
              wx1_ref, bx1_ref, wx2_ref, bx2_ref,
              wh1_ref, bh1_ref, wh2_ref, bh2_ref,
              score_ref, attr_ref, neigh_ref):
    hp = lambda a, b: jnp.dot(a, b, preferred_element_type=jnp.float32,
                              precision=lax.Precision.HIGHEST)
    h = h_ref[...]
    h_bar = (s2_ref[0] + s2_ref[1] + h) * ic_ref[...]
    x_hat = hp(jnp.maximum(hp(h, wx1_ref[...]) + bx1_ref[...], 0.0),
               wx2_ref[...]) + bx2_ref[...]
    m_hat = hp(jnp.maximum(hp(h_bar, wh1_ref[...]) + bh1_ref[...], 0.0),
               wh2_ref[...]) + bh2_ref[...]
    x = x_ref[...]
    attr = jnp.sqrt(jnp.sum((x_hat - x) ** 2, axis=1, keepdims=True) + 1e-12)
    neigh = jnp.sqrt(jnp.sum((m_hat - m_ref[...]) ** 2, axis=1,
                             keepdims=True) + 1e-12)
    homo = jnp.sqrt(jnp.sum((m_hat - x) ** 2, axis=1, keepdims=True) + 1e-12)
    score_ref[...] = attr + neigh + 0.5 * homo
    attr_ref[...] = attr
    neigh_ref[...] = neigh


_ROWS = 2000


def kernel(x, edge_index, edge_weight, W_enc, b_enc,
           Wx1, bx1, Wx2, bx2, Wh1, bh1, Wh2, bh2):
    del edge_weight
    src = edge_index[0]
    dst = edge_index[1]

    x_aug = jnp.concatenate(
        [x, jnp.ones((N, 1), jnp.float32), jnp.zeros((N, AUG - F_DIM - 1),
                                                     jnp.float32)], axis=1)

    p1 = _seg_sum_aug(x_aug, src, dst, jnp.zeros((NPAD, AUG), jnp.float32))

    grid = N // _ROWS
    m, h, inv_cnt = pl.pallas_call(
        _enc_body,
        grid=(grid,),
        in_specs=[
            pl.BlockSpec((NC, _ROWS, AUG), lambda i: (0, i, 0)),
            pl.BlockSpec((_ROWS, F_DIM), lambda i: (i, 0)),
            pl.BlockSpec((F_DIM, H_DIM), lambda i: (0, 0)),
            pl.BlockSpec((1, H_DIM), lambda i: (0, 0)),
        ],
        out_specs=[
            pl.BlockSpec((_ROWS, F_DIM), lambda i: (i, 0)),
            pl.BlockSpec((_ROWS, H_DIM), lambda i: (i, 0)),
            pl.BlockSpec((_ROWS, 1), lambda i: (i, 0)),
        ],
        out_shape=[
            jax.ShapeDtypeStruct((N, F_DIM), jnp.float32),
            jax.ShapeDtypeStruct((N, H_DIM), jnp.float32),
            jax.ShapeDtypeStruct((N, 1), jnp.float32),
        ],
    )(p1, x, W_enc, b_enc.reshape(1, H_DIM))

    p2 = _seg_sum_h(h, src, dst, jnp.zeros((NPAD, H_DIM), jnp.float32))

    score, attr_err, neigh_err = pl.pallas_call(
        _dec_body,
        grid=(grid,),
        in_specs=[
            pl.BlockSpec((NC, _ROWS, H_DIM), lambda i: (0, i, 0)),
            pl.BlockSpec((_ROWS, H_DIM), lambda i: (i, 0)),
            pl.BlockSpec((_ROWS, F_DIM), lambda i: (i, 0)),
            pl.BlockSpec((_ROWS, F_DIM), lambda i: (i, 0)),
            pl.BlockSpec((_ROWS, 1), lambda i: (i, 0)),
            pl.BlockSpec((H_DIM, DEC_H), lambda i: (0, 0)),
            pl.BlockSpec((1, DEC_H), lambda i: (0, 0)),
            pl.BlockSpec((DEC_H, F_DIM), lambda i: (0, 0)),
            pl.BlockSpec((1, F_DIM), lambda i: (0, 0)),
            pl.BlockSpec((H_DIM, DEC_H), lambda i: (0, 0)),
            pl.BlockSpec((1, DEC_H), lambda i: (0, 0)),
            pl.BlockSpec((DEC_H, F_DIM), lambda i: (0, 0)),
            pl.BlockSpec((1, F_DIM), lambda i: (0, 0)),
        ],
        out_specs=[
            pl.BlockSpec((_ROWS, 1), lambda i: (i, 0)),
            pl.BlockSpec((_ROWS, 1), lambda i: (i, 0)),
            pl.BlockSpec((_ROWS, 1), lambda i: (i, 0)),
        ],
        out_shape=[
            jax.ShapeDtypeStruct((N, 1), jnp.float32),
            jax.ShapeDtypeStruct((N, 1), jnp.float32),
            jax.ShapeDtypeStruct((N, 1), jnp.float32),
        ],
    )(p2, h, x, m, inv_cnt,
      Wx1, bx1.reshape(1, DEC_H), Wx2, bx2.reshape(1, F_DIM),
      Wh1, bh1.reshape(1, DEC_H), Wh2, bh2.reshape(1, F_DIM))

    return (score[:, 0], attr_err[:, 0], neigh_err[:, 0])


# speedup vs baseline: 15.0572x; 1.9046x over previous
import functools

import jax
import jax.numpy as jnp
from jax import lax
from jax.experimental import pallas as pl
from jax.experimental.pallas import tpu as pltpu
from jax.experimental.pallas import tpu_sc as plsc

N = 10000
NPAD = 10000
E = 320000
F_DIM = 128
H_DIM = 64
DEC_H = 128
AUG = 144

NC = 2
NS = 16
NW = NC * NS
CHUNK = 64
NCHUNKS = E // CHUNK
CPW = NCHUNKS // NW
CPW_MAX = CPW + 1
EXTRA = NCHUNKS - CPW * NW


def _seg_sum_sc(d):
    rows_per_sub = NPAD // NS

    mesh = plsc.VectorSubcoreMesh(
        core_axis_name="c", subcore_axis_name="s",
        num_cores=NC, num_subcores=NS)

    @functools.partial(
        pl.kernel,
        out_type=jax.ShapeDtypeStruct((NC, NPAD, d), jnp.float32),
        mesh=mesh,
        scratch_types=[
            pltpu.VMEM((CPW_MAX, CHUNK), jnp.int32),
            pltpu.VMEM((CPW_MAX, CHUNK), jnp.int32),
            pltpu.VMEM((CHUNK, d), jnp.float32),
            pltpu.VMEM((CHUNK, d), jnp.float32),
            pltpu.VMEM_SHARED((NPAD, d), jnp.float32),
            pltpu.SemaphoreType.DMA,
            pltpu.SemaphoreType.DMA,
        ],
        compiler_params=pltpu.CompilerParams(use_tc_tiling_on_sc=False),
    )
    def seg_sum(table_hbm, src_hbm, dst_hbm, zeros_hbm, out_hbm,
                src_v, dst_v, buf0, buf1, acc_sh, sem0, sem1):
        c = lax.axis_index("c")
        s = lax.axis_index("s")
        w = c * NS + s
        row0 = s * rows_per_sub
        start = w * CPW + jnp.minimum(w, EXTRA)
        n_i = CPW + jnp.where(w < EXTRA, 1, 0)

        pltpu.sync_copy(src_hbm.at[pl.ds(start, CPW_MAX)], src_v)
        pltpu.sync_copy(dst_hbm.at[pl.ds(start, CPW_MAX)], dst_v)
        pltpu.sync_copy(zeros_hbm, acc_sh.at[pl.ds(row0, rows_per_sub)])
        plsc.subcore_barrier()

        bufs = (buf0, buf1)
        sems = (sem0, sem1)
        for b in range(2):
            pltpu.async_copy(table_hbm.at[src_v.at[b]], bufs[b], sems[b])

        def body(g, _):
            for b in range(2):
                k = 2 * g + b

                @pl.when(k < n_i)
                def _():
                    pltpu.make_async_copy(
                        table_hbm.at[src_v.at[k]], bufs[b], sems[b]).wait()
                    pltpu.sync_copy(bufs[b], acc_sh.at[dst_v.at[k]], add=True)

                    @pl.when(k + 2 < n_i)
                    def _():
                        pltpu.async_copy(
                            table_hbm.at[src_v.at[k + 2]], bufs[b], sems[b])
            return 0

        lax.fori_loop(0, CPW_MAX // 2 + 1, body, 0)
        plsc.subcore_barrier()
        pltpu.sync_copy(acc_sh.at[pl.ds(row0, rows_per_sub)],
                        out_hbm.at[c, pl.ds(row0, rows_per_sub)])

    return seg_sum


_seg_sum_aug = _seg_sum_sc(AUG)
_seg_sum_h = _seg_sum_sc(H_DIM)


def _enc_body(p_ref, x_ref, w_ref, b_ref, m_ref, h_ref, ic_ref):
    ssum = p_ref[0] + p_ref[1]
    inv = 1.0 / (ssum[:, 128:129] + 1.0)
    m = (ssum[:, :F_DIM] + x_ref[...]) * inv
    m_ref[...] = m
    h_ref[...] = jnp.maximum(
        jnp.dot(m, w_ref[...], preferred_element_type=jnp.float32,
                precision=lax.Precision.HIGHEST) + b_ref[...], 0.0)
    ic_ref[...] = inv


def _dec_body(s2_ref, h_ref, x_ref, m_ref, ic_ref,
              wx1_ref, bx1_ref, wx2_ref, bx2_ref,
              wh1_ref, bh1_ref, wh2_ref, bh2_ref,
              score_ref, attr_ref, neigh_ref):
    hp = lambda a, b: jnp.dot(a, b, preferred_element_type=jnp.float32,
                              precision=lax.Precision.HIGHEST)
    h = h_ref[...]
    h_bar = (s2_ref[0] + s2_ref[1] + h) * ic_ref[...]
    x_hat = hp(jnp.maximum(hp(h, wx1_ref[...]) + bx1_ref[...], 0.0),
               wx2_ref[...]) + bx2_ref[...]
    m_hat = hp(jnp.maximum(hp(h_bar, wh1_ref[...]) + bh1_ref[...], 0.0),
               wh2_ref[...]) + bh2_ref[...]
    x = x_ref[...]
    attr = jnp.sqrt(jnp.sum((x_hat - x) ** 2, axis=1, keepdims=True) + 1e-12)
    neigh = jnp.sqrt(jnp.sum((m_hat - m_ref[...]) ** 2, axis=1,
                             keepdims=True) + 1e-12)
    homo = jnp.sqrt(jnp.sum((m_hat - x) ** 2, axis=1, keepdims=True) + 1e-12)
    score_ref[...] = attr + neigh + 0.5 * homo
    attr_ref[...] = attr
    neigh_ref[...] = neigh


_ROWS = 2000


def kernel(x, edge_index, edge_weight, W_enc, b_enc,
           Wx1, bx1, Wx2, bx2, Wh1, bh1, Wh2, bh2):
    del edge_weight
    pad = jnp.zeros((2, CHUNK), jnp.int32)
    ei = jnp.concatenate([edge_index, pad], axis=1)
    src = ei[0].reshape(NCHUNKS + 1, CHUNK)
    dst = ei[1].reshape(NCHUNKS + 1, CHUNK)

    x_aug = jnp.concatenate(
        [x, jnp.ones((N, 1), jnp.float32), jnp.zeros((N, AUG - F_DIM - 1),
                                                     jnp.float32)], axis=1)

    p1 = _seg_sum_aug(x_aug, src, dst,
                      jnp.zeros((NPAD // NS, AUG), jnp.float32))

    grid = N // _ROWS
    m, h, inv_cnt = pl.pallas_call(
        _enc_body,
        grid=(grid,),
        in_specs=[
            pl.BlockSpec((NC, _ROWS, AUG), lambda i: (0, i, 0)),
            pl.BlockSpec((_ROWS, F_DIM), lambda i: (i, 0)),
            pl.BlockSpec((F_DIM, H_DIM), lambda i: (0, 0)),
            pl.BlockSpec((1, H_DIM), lambda i: (0, 0)),
        ],
        out_specs=[
            pl.BlockSpec((_ROWS, F_DIM), lambda i: (i, 0)),
            pl.BlockSpec((_ROWS, H_DIM), lambda i: (i, 0)),
            pl.BlockSpec((_ROWS, 1), lambda i: (i, 0)),
        ],
        out_shape=[
            jax.ShapeDtypeStruct((N, F_DIM), jnp.float32),
            jax.ShapeDtypeStruct((N, H_DIM), jnp.float32),
            jax.ShapeDtypeStruct((N, 1), jnp.float32),
        ],
    )(p1, x, W_enc, b_enc.reshape(1, H_DIM))

    p2 = _seg_sum_h(h, src, dst, jnp.zeros((NPAD // NS, H_DIM), jnp.float32))

    score, attr_err, neigh_err = pl.pallas_call(
        _dec_body,
        grid=(grid,),
        in_specs=[
            pl.BlockSpec((NC, _ROWS, H_DIM), lambda i: (0, i, 0)),
            pl.BlockSpec((_ROWS, H_DIM), lambda i: (i, 0)),
            pl.BlockSpec((_ROWS, F_DIM), lambda i: (i, 0)),
            pl.BlockSpec((_ROWS, F_DIM), lambda i: (i, 0)),
            pl.BlockSpec((_ROWS, 1), lambda i: (i, 0)),
            pl.BlockSpec((H_DIM, DEC_H), lambda i: (0, 0)),
            pl.BlockSpec((1, DEC_H), lambda i: (0, 0)),
            pl.BlockSpec((DEC_H, F_DIM), lambda i: (0, 0)),
            pl.BlockSpec((1, F_DIM), lambda i: (0, 0)),
            pl.BlockSpec((H_DIM, DEC_H), lambda i: (0, 0)),
            pl.BlockSpec((1, DEC_H), lambda i: (0, 0)),
            pl.BlockSpec((DEC_H, F_DIM), lambda i: (0, 0)),
            pl.BlockSpec((1, F_DIM), lambda i: (0, 0)),
        ],
        out_specs=[
            pl.BlockSpec((_ROWS, 1), lambda i: (i, 0)),
            pl.BlockSpec((_ROWS, 1), lambda i: (i, 0)),
            pl.BlockSpec((_ROWS, 1), lambda i: (i, 0)),
        ],
        out_shape=[
            jax.ShapeDtypeStruct((N, 1), jnp.float32),
            jax.ShapeDtypeStruct((N, 1), jnp.float32),
            jax.ShapeDtypeStruct((N, 1), jnp.float32),
        ],
    )(p2, h, x, m, inv_cnt,
      Wx1, bx1.reshape(1, DEC_H), Wx2, bx2.reshape(1, F_DIM),
      Wh1, bh1.reshape(1, DEC_H), Wh2, bh2.reshape(1, F_DIM))

    return (score[:, 0], attr_err[:, 0], neigh_err[:, 0])


# speedup vs baseline: 15.7177x; 1.0439x over previous
import functools

import jax
import jax.numpy as jnp
from jax import lax
from jax.experimental import pallas as pl
from jax.experimental.pallas import tpu as pltpu
from jax.experimental.pallas import tpu_sc as plsc

N = 10000
NPAD = 10000
E = 320000
F_DIM = 128
H_DIM = 64
DEC_H = 128
AUG = 144

NC = 2
NS = 16
NW = NC * NS
CHUNK = 128
NCHUNKS = E // CHUNK
CPW = NCHUNKS // NW
CPW_MAX = CPW + 1
EXTRA = NCHUNKS - CPW * NW


def _seg_sum_sc(d):
    rows_per_sub = NPAD // NS

    mesh = plsc.VectorSubcoreMesh(
        core_axis_name="c", subcore_axis_name="s",
        num_cores=NC, num_subcores=NS)

    @functools.partial(
        pl.kernel,
        out_type=jax.ShapeDtypeStruct((NC, NPAD, d), jnp.float32),
        mesh=mesh,
        scratch_types=[
            pltpu.VMEM((CHUNK,), jnp.int32),
            pltpu.VMEM((CHUNK,), jnp.int32),
            pltpu.VMEM((CHUNK,), jnp.int32),
            pltpu.VMEM((CHUNK,), jnp.int32),
            pltpu.VMEM((CHUNK, d), jnp.float32),
            pltpu.VMEM((CHUNK, d), jnp.float32),
            pltpu.VMEM_SHARED((NPAD, d), jnp.float32),
            pltpu.SemaphoreType.DMA,
            pltpu.SemaphoreType.DMA,
            pltpu.SemaphoreType.DMA,
            pltpu.SemaphoreType.DMA,
            pltpu.SemaphoreType.DMA,
            pltpu.SemaphoreType.DMA,
        ],
        compiler_params=pltpu.CompilerParams(use_tc_tiling_on_sc=False),
    )
    def seg_sum(table_hbm, src_hbm, dst_hbm, zeros_hbm, out_hbm,
                si0, si1, di0, di1, buf0, buf1, acc_sh,
                ss0, ss1, ds0, ds1, gs0, gs1):
        c = lax.axis_index("c")
        s = lax.axis_index("s")
        w = c * NS + s
        row0 = s * rows_per_sub
        start = w * CPW + jnp.minimum(w, EXTRA)
        n_i = CPW + jnp.where(w < EXTRA, 1, 0)

        sidx = (si0, si1)
        didx = (di0, di1)
        bufs = (buf0, buf1)
        ssem = (ss0, ss1)
        dsem = (ds0, ds1)
        gsem = (gs0, gs1)

        pltpu.async_copy(src_hbm.at[start], si0, ss0)
        pltpu.async_copy(dst_hbm.at[start], di0, ds0)
        pltpu.async_copy(src_hbm.at[start + 1], si1, ss1)
        pltpu.async_copy(dst_hbm.at[start + 1], di1, ds1)
        pltpu.sync_copy(zeros_hbm, acc_sh.at[pl.ds(row0, rows_per_sub)])
        plsc.subcore_barrier()
        pltpu.make_async_copy(src_hbm.at[start], si0, ss0).wait()
        pltpu.async_copy(table_hbm.at[si0], buf0, gs0)

        def step(k, b):
            b1 = 1 - b
            kk = start + k

            @pl.when(k + 1 < n_i)
            def _():
                pltpu.make_async_copy(
                    src_hbm.at[kk + 1], sidx[b1], ssem[b1]).wait()
                pltpu.async_copy(table_hbm.at[sidx[b1]], bufs[b1], gsem[b1])

            @pl.when(k < n_i)
            def _():
                pltpu.make_async_copy(
                    table_hbm.at[sidx[b]], bufs[b], gsem[b]).wait()
                pltpu.make_async_copy(dst_hbm.at[kk], didx[b], dsem[b]).wait()
                pltpu.sync_copy(bufs[b], acc_sh.at[didx[b]], add=True)

                @pl.when(k + 2 < n_i)
                def _():
                    pltpu.async_copy(src_hbm.at[kk + 2], sidx[b], ssem[b])
                    pltpu.async_copy(dst_hbm.at[kk + 2], didx[b], dsem[b])

        def body(g, _):
            step(2 * g, 0)
            step(2 * g + 1, 1)
            return 0

        lax.fori_loop(0, CPW_MAX // 2 + 1, body, 0)
        plsc.subcore_barrier()
        pltpu.sync_copy(acc_sh.at[pl.ds(row0, rows_per_sub)],
                        out_hbm.at[c, pl.ds(row0, rows_per_sub)])

    return seg_sum


_seg_sum_aug = _seg_sum_sc(AUG)
_seg_sum_h = _seg_sum_sc(H_DIM)


def _enc_body(p_ref, x_ref, w_ref, b_ref, m_ref, h_ref, ic_ref):
    ssum = p_ref[0] + p_ref[1]
    inv = 1.0 / (ssum[:, 128:129] + 1.0)
    m = (ssum[:, :F_DIM] + x_ref[...]) * inv
    m_ref[...] = m
    h_ref[...] = jnp.maximum(
        jnp.dot(m, w_ref[...], preferred_element_type=jnp.float32,
                precision=lax.Precision.HIGHEST) + b_ref[...], 0.0)
    ic_ref[...] = inv


def _dec_body(s2_ref, h_ref, x_ref, m_ref, ic_ref,
              wx1_ref, bx1_ref, wx2_ref, bx2_ref,
              wh1_ref, bh1_ref, wh2_ref, bh2_ref,
              score_ref, attr_ref, neigh_ref):
    hp = lambda a, b: jnp.dot(a, b, preferred_element_type=jnp.float32,
                              precision=lax.Precision.HIGHEST)
    h = h_ref[...]
    h_bar = (s2_ref[0] + s2_ref[1] + h) * ic_ref[...]
    x_hat = hp(jnp.maximum(hp(h, wx1_ref[...]) + bx1_ref[...], 0.0),
               wx2_ref[...]) + bx2_ref[...]
    m_hat = hp(jnp.maximum(hp(h_bar, wh1_ref[...]) + bh1_ref[...], 0.0),
               wh2_ref[...]) + bh2_ref[...]
    x = x_ref[...]
    attr = jnp.sqrt(jnp.sum((x_hat - x) ** 2, axis=1, keepdims=True) + 1e-12)
    neigh = jnp.sqrt(jnp.sum((m_hat - m_ref[...]) ** 2, axis=1,
                             keepdims=True) + 1e-12)
    homo = jnp.sqrt(jnp.sum((m_hat - x) ** 2, axis=1, keepdims=True) + 1e-12)
    score_ref[...] = attr + neigh + 0.5 * homo
    attr_ref[...] = attr
    neigh_ref[...] = neigh


_ROWS = 2000


def kernel(x, edge_index, edge_weight, W_enc, b_enc,
           Wx1, bx1, Wx2, bx2, Wh1, bh1, Wh2, bh2):
    del edge_weight
    pad = jnp.zeros((2, CHUNK), jnp.int32)
    ei = jnp.concatenate([edge_index, pad], axis=1)
    src = ei[0].reshape(NCHUNKS + 1, CHUNK)
    dst = ei[1].reshape(NCHUNKS + 1, CHUNK)

    x_aug = jnp.concatenate(
        [x, jnp.ones((N, 1), jnp.float32), jnp.zeros((N, AUG - F_DIM - 1),
                                                     jnp.float32)], axis=1)

    p1 = _seg_sum_aug(x_aug, src, dst,
                      jnp.zeros((NPAD // NS, AUG), jnp.float32))

    grid = N // _ROWS
    m, h, inv_cnt = pl.pallas_call(
        _enc_body,
        grid=(grid,),
        in_specs=[
            pl.BlockSpec((NC, _ROWS, AUG), lambda i: (0, i, 0)),
            pl.BlockSpec((_ROWS, F_DIM), lambda i: (i, 0)),
            pl.BlockSpec((F_DIM, H_DIM), lambda i: (0, 0)),
            pl.BlockSpec((1, H_DIM), lambda i: (0, 0)),
        ],
        out_specs=[
            pl.BlockSpec((_ROWS, F_DIM), lambda i: (i, 0)),
            pl.BlockSpec((_ROWS, H_DIM), lambda i: (i, 0)),
            pl.BlockSpec((_ROWS, 1), lambda i: (i, 0)),
        ],
        out_shape=[
            jax.ShapeDtypeStruct((N, F_DIM), jnp.float32),
            jax.ShapeDtypeStruct((N, H_DIM), jnp.float32),
            jax.ShapeDtypeStruct((N, 1), jnp.float32),
        ],
    )(p1, x, W_enc, b_enc.reshape(1, H_DIM))

    p2 = _seg_sum_h(h, src, dst, jnp.zeros((NPAD // NS, H_DIM), jnp.float32))

    score, attr_err, neigh_err = pl.pallas_call(
        _dec_body,
        grid=(grid,),
        in_specs=[
            pl.BlockSpec((NC, _ROWS, H_DIM), lambda i: (0, i, 0)),
            pl.BlockSpec((_ROWS, H_DIM), lambda i: (i, 0)),
            pl.BlockSpec((_ROWS, F_DIM), lambda i: (i, 0)),
            pl.BlockSpec((_ROWS, F_DIM), lambda i: (i, 0)),
            pl.BlockSpec((_ROWS, 1), lambda i: (i, 0)),
            pl.BlockSpec((H_DIM, DEC_H), lambda i: (0, 0)),
            pl.BlockSpec((1, DEC_H), lambda i: (0, 0)),
            pl.BlockSpec((DEC_H, F_DIM), lambda i: (0, 0)),
            pl.BlockSpec((1, F_DIM), lambda i: (0, 0)),
            pl.BlockSpec((H_DIM, DEC_H), lambda i: (0, 0)),
            pl.BlockSpec((1, DEC_H), lambda i: (0, 0)),
            pl.BlockSpec((DEC_H, F_DIM), lambda i: (0, 0)),
            pl.BlockSpec((1, F_DIM), lambda i: (0, 0)),
        ],
        out_specs=[
            pl.BlockSpec((_ROWS, 1), lambda i: (i, 0)),
            pl.BlockSpec((_ROWS, 1), lambda i: (i, 0)),
            pl.BlockSpec((_ROWS, 1), lambda i: (i, 0)),
        ],
        out_shape=[
            jax.ShapeDtypeStruct((N, 1), jnp.float32),
            jax.ShapeDtypeStruct((N, 1), jnp.float32),
            jax.ShapeDtypeStruct((N, 1), jnp.float32),
        ],
    )(p2, h, x, m, inv_cnt,
      Wx1, bx1.reshape(1, DEC_H), Wx2, bx2.reshape(1, F_DIM),
      Wh1, bh1.reshape(1, DEC_H), Wh2, bh2.reshape(1, F_DIM))

    return (score[:, 0], attr_err[:, 0], neigh_err[:, 0])


# speedup vs baseline: 16.2967x; 1.0368x over previous
import functools

import jax
import jax.numpy as jnp
from jax import lax
from jax.experimental import pallas as pl
from jax.experimental.pallas import tpu as pltpu
from jax.experimental.pallas import tpu_sc as plsc

N = 10000
NPAD = 10000
E = 320000
F_DIM = 128
H_DIM = 64
DEC_H = 128
AUG = 144

NC = 2
NS = 16
NW = NC * NS
CHUNK = 128
NCHUNKS = E // CHUNK
CPW = NCHUNKS // NW
CPW_MAX = CPW + 1
EXTRA = NCHUNKS - CPW * NW


def _seg_sum_sc(d):
    rows_per_sub = NPAD // NS

    mesh = plsc.VectorSubcoreMesh(
        core_axis_name="c", subcore_axis_name="s",
        num_cores=NC, num_subcores=NS)

    @functools.partial(
        pl.kernel,
        out_type=jax.ShapeDtypeStruct((NC, NPAD, d), jnp.float32),
        mesh=mesh,
        scratch_types=[
            pltpu.VMEM((CHUNK,), jnp.int32),
            pltpu.VMEM((CHUNK,), jnp.int32),
            pltpu.VMEM((CHUNK,), jnp.int32),
            pltpu.VMEM((CHUNK,), jnp.int32),
            pltpu.VMEM((CHUNK, d), jnp.float32),
            pltpu.VMEM((CHUNK, d), jnp.float32),
            pltpu.VMEM_SHARED((NPAD, d), jnp.float32),
            pltpu.SemaphoreType.DMA,
            pltpu.SemaphoreType.DMA,
            pltpu.SemaphoreType.DMA,
            pltpu.SemaphoreType.DMA,
            pltpu.SemaphoreType.DMA,
            pltpu.SemaphoreType.DMA,
        ],
        compiler_params=pltpu.CompilerParams(use_tc_tiling_on_sc=False),
    )
    def seg_sum(table_hbm, src_hbm, dst_hbm, zeros_hbm, out_hbm,
                si0, si1, di0, di1, buf0, buf1, acc_sh,
                ss0, ss1, ds0, ds1, gs0, gs1):
        c = lax.axis_index("c")
        s = lax.axis_index("s")
        w = c * NS + s
        row0 = s * rows_per_sub
        start = w * CPW + jnp.minimum(w, EXTRA)
        n_i = CPW + jnp.where(w < EXTRA, 1, 0)

        sidx = (si0, si1)
        didx = (di0, di1)
        bufs = (buf0, buf1)
        ssem = (ss0, ss1)
        dsem = (ds0, ds1)
        gsem = (gs0, gs1)

        pltpu.async_copy(src_hbm.at[start], si0, ss0)
        pltpu.async_copy(dst_hbm.at[start], di0, ds0)
        pltpu.async_copy(src_hbm.at[start + 1], si1, ss1)
        pltpu.async_copy(dst_hbm.at[start + 1], di1, ds1)
        pltpu.sync_copy(zeros_hbm, acc_sh.at[pl.ds(row0, rows_per_sub)])
        plsc.subcore_barrier()
        pltpu.make_async_copy(src_hbm.at[start], si0, ss0).wait()
        pltpu.async_copy(table_hbm.at[si0], buf0, gs0)

        def step(k, b):
            b1 = 1 - b
            kk = start + k

            @pl.when(k + 1 < n_i)
            def _():
                pltpu.make_async_copy(
                    src_hbm.at[kk + 1], sidx[b1], ssem[b1]).wait()
                pltpu.async_copy(table_hbm.at[sidx[b1]], bufs[b1], gsem[b1])

            @pl.when(k < n_i)
            def _():
                pltpu.make_async_copy(
                    table_hbm.at[sidx[b]], bufs[b], gsem[b]).wait()
                pltpu.make_async_copy(dst_hbm.at[kk], didx[b], dsem[b]).wait()
                pltpu.sync_copy(bufs[b], acc_sh.at[didx[b]], add=True)

                @pl.when(k + 2 < n_i)
                def _():
                    pltpu.async_copy(src_hbm.at[kk + 2], sidx[b], ssem[b])
                    pltpu.async_copy(dst_hbm.at[kk + 2], didx[b], dsem[b])

        def body(g, _):
            step(2 * g, 0)
            step(2 * g + 1, 1)
            return 0

        lax.fori_loop(0, CPW_MAX // 2 + 1, body, 0)
        plsc.subcore_barrier()
        pltpu.sync_copy(acc_sh.at[pl.ds(row0, rows_per_sub)],
                        out_hbm.at[c, pl.ds(row0, rows_per_sub)])

    return seg_sum


_seg_sum_aug = _seg_sum_sc(AUG)
_seg_sum_h = _seg_sum_sc(H_DIM)


def _hp(a, b):
    return jnp.dot(a, b, preferred_element_type=jnp.float32,
                   precision=lax.Precision.DEFAULT)


def _enc_body(p_ref, x_ref, w_ref, b_ref, m_ref, h_ref, ic_ref):
    ssum = p_ref[0] + p_ref[1]
    inv = 1.0 / (ssum[:, 128:129] + 1.0)
    m = (ssum[:, :F_DIM] + x_ref[...]) * inv
    m_ref[...] = m
    h_ref[...] = jnp.maximum(_hp(m, w_ref[...]) + b_ref[...], 0.0)
    ic_ref[...] = inv


def _deca_body(h_ref, x_ref, wx1_ref, bx1_ref, wx2_ref, bx2_ref, attr_ref):
    x_hat = _hp(jnp.maximum(_hp(h_ref[...], wx1_ref[...]) + bx1_ref[...],
                            0.0), wx2_ref[...]) + bx2_ref[...]
    attr_ref[...] = jnp.sqrt(
        jnp.sum((x_hat - x_ref[...]) ** 2, axis=1, keepdims=True) + 1e-12)


def _decb_body(s2_ref, h_ref, x_ref, m_ref, ic_ref, attr_ref,
               wh1_ref, bh1_ref, wh2_ref, bh2_ref,
               score_ref, attr_o_ref, neigh_ref):
    h_bar = (s2_ref[0] + s2_ref[1] + h_ref[...]) * ic_ref[...]
    m_hat = _hp(jnp.maximum(_hp(h_bar, wh1_ref[...]) + bh1_ref[...], 0.0),
                wh2_ref[...]) + bh2_ref[...]
    x = x_ref[...]
    attr = attr_ref[...]
    neigh = jnp.sqrt(jnp.sum((m_hat - m_ref[...]) ** 2, axis=1,
                             keepdims=True) + 1e-12)
    homo = jnp.sqrt(jnp.sum((m_hat - x) ** 2, axis=1, keepdims=True) + 1e-12)
    score_ref[...] = attr + neigh + 0.5 * homo
    attr_o_ref[...] = attr
    neigh_ref[...] = neigh


_ROWS = 2000


def kernel(x, edge_index, edge_weight, W_enc, b_enc,
           Wx1, bx1, Wx2, bx2, Wh1, bh1, Wh2, bh2):
    del edge_weight
    pad = jnp.zeros((2, CHUNK), jnp.int32)
    ei = jnp.concatenate([edge_index, pad], axis=1)
    src = ei[0].reshape(NCHUNKS + 1, CHUNK)
    dst = ei[1].reshape(NCHUNKS + 1, CHUNK)

    x_aug = jnp.concatenate(
        [x, jnp.ones((N, 1), jnp.float32), jnp.zeros((N, AUG - F_DIM - 1),
                                                     jnp.float32)], axis=1)

    p1 = _seg_sum_aug(x_aug, src, dst,
                      jnp.zeros((NPAD // NS, AUG), jnp.float32))

    grid = N // _ROWS
    m, h, inv_cnt = pl.pallas_call(
        _enc_body,
        grid=(grid,),
        in_specs=[
            pl.BlockSpec((NC, _ROWS, AUG), lambda i: (0, i, 0)),
            pl.BlockSpec((_ROWS, F_DIM), lambda i: (i, 0)),
            pl.BlockSpec((F_DIM, H_DIM), lambda i: (0, 0)),
            pl.BlockSpec((1, H_DIM), lambda i: (0, 0)),
        ],
        out_specs=[
            pl.BlockSpec((_ROWS, F_DIM), lambda i: (i, 0)),
            pl.BlockSpec((_ROWS, H_DIM), lambda i: (i, 0)),
            pl.BlockSpec((_ROWS, 1), lambda i: (i, 0)),
        ],
        out_shape=[
            jax.ShapeDtypeStruct((N, F_DIM), jnp.float32),
            jax.ShapeDtypeStruct((N, H_DIM), jnp.float32),
            jax.ShapeDtypeStruct((N, 1), jnp.float32),
        ],
    )(p1, x, W_enc, b_enc.reshape(1, H_DIM))

    p2 = _seg_sum_h(h, src, dst, jnp.zeros((NPAD // NS, H_DIM), jnp.float32))

    attr1 = pl.pallas_call(
        _deca_body,
        grid=(grid,),
        in_specs=[
            pl.BlockSpec((_ROWS, H_DIM), lambda i: (i, 0)),
            pl.BlockSpec((_ROWS, F_DIM), lambda i: (i, 0)),
            pl.BlockSpec((H_DIM, DEC_H), lambda i: (0, 0)),
            pl.BlockSpec((1, DEC_H), lambda i: (0, 0)),
            pl.BlockSpec((DEC_H, F_DIM), lambda i: (0, 0)),
            pl.BlockSpec((1, F_DIM), lambda i: (0, 0)),
        ],
        out_specs=pl.BlockSpec((_ROWS, 1), lambda i: (i, 0)),
        out_shape=jax.ShapeDtypeStruct((N, 1), jnp.float32),
    )(h, x, Wx1, bx1.reshape(1, DEC_H), Wx2, bx2.reshape(1, F_DIM))

    score, attr_err, neigh_err = pl.pallas_call(
        _decb_body,
        grid=(grid,),
        in_specs=[
            pl.BlockSpec((NC, _ROWS, H_DIM), lambda i: (0, i, 0)),
            pl.BlockSpec((_ROWS, H_DIM), lambda i: (i, 0)),
            pl.BlockSpec((_ROWS, F_DIM), lambda i: (i, 0)),
            pl.BlockSpec((_ROWS, F_DIM), lambda i: (i, 0)),
            pl.BlockSpec((_ROWS, 1), lambda i: (i, 0)),
            pl.BlockSpec((_ROWS, 1), lambda i: (i, 0)),
            pl.BlockSpec((H_DIM, DEC_H), lambda i: (0, 0)),
            pl.BlockSpec((1, DEC_H), lambda i: (0, 0)),
            pl.BlockSpec((DEC_H, F_DIM), lambda i: (0, 0)),
            pl.BlockSpec((1, F_DIM), lambda i: (0, 0)),
        ],
        out_specs=[
            pl.BlockSpec((_ROWS, 1), lambda i: (i, 0)),
            pl.BlockSpec((_ROWS, 1), lambda i: (i, 0)),
            pl.BlockSpec((_ROWS, 1), lambda i: (i, 0)),
        ],
        out_shape=[
            jax.ShapeDtypeStruct((N, 1), jnp.float32),
            jax.ShapeDtypeStruct((N, 1), jnp.float32),
            jax.ShapeDtypeStruct((N, 1), jnp.float32),
        ],
    )(p2, h, x, m, inv_cnt, attr1,
      Wh1, bh1.reshape(1, DEC_H), Wh2, bh2.reshape(1, F_DIM))

    return (score[:, 0], attr_err[:, 0], neigh_err[:, 0])


# speedup vs baseline: 17.8224x; 1.0936x over previous
import functools

import jax
import jax.numpy as jnp
from jax import lax
from jax.experimental import pallas as pl
from jax.experimental.pallas import tpu as pltpu
from jax.experimental.pallas import tpu_sc as plsc

N = 10000
NPAD = 10000
E = 320000
F_DIM = 128
H_DIM = 64
DEC_H = 128

NC = 2
NS = 16
NW = NC * NS
CHUNK = 128
NCHUNKS = E // CHUNK
CPW = NCHUNKS // NW
CPW_MAX = CPW + 1
EXTRA = NCHUNKS - CPW * NW


def _seg_sum_sc(d, with_counts):
    rows_per_sub = NPAD // NS

    mesh = plsc.VectorSubcoreMesh(
        core_axis_name="c", subcore_axis_name="s",
        num_cores=NC, num_subcores=NS)

    out_type = [jax.ShapeDtypeStruct((NC, NPAD, d), jnp.float32)]
    scratch = [
        pltpu.VMEM((CHUNK,), jnp.int32),
        pltpu.VMEM((CHUNK,), jnp.int32),
        pltpu.VMEM((CHUNK,), jnp.int32),
        pltpu.VMEM((CHUNK,), jnp.int32),
        pltpu.VMEM((CHUNK, d), jnp.float32),
        pltpu.VMEM((CHUNK, d), jnp.float32),
        pltpu.VMEM_SHARED((NPAD, d), jnp.float32),
        pltpu.SemaphoreType.DMA,
        pltpu.SemaphoreType.DMA,
        pltpu.SemaphoreType.DMA,
        pltpu.SemaphoreType.DMA,
        pltpu.SemaphoreType.DMA,
        pltpu.SemaphoreType.DMA,
    ]
    if with_counts:
        out_type.append(jax.ShapeDtypeStruct((NC, NPAD, 16), jnp.float32))
        scratch += [
            pltpu.VMEM((CHUNK, 16), jnp.float32),
            pltpu.VMEM_SHARED((NPAD, 16), jnp.float32),
        ]

    @functools.partial(
        pl.kernel,
        out_type=out_type,
        mesh=mesh,
        scratch_types=scratch,
        compiler_params=pltpu.CompilerParams(use_tc_tiling_on_sc=False),
    )
    def seg_sum(table_hbm, src_hbm, dst_hbm, zeros_hbm, *rest):
        if with_counts:
            (ones_hbm, out_hbm, cnt_hbm,
             si0, si1, di0, di1, buf0, buf1, acc_sh,
             ss0, ss1, ds0, ds1, gs0, gs1, ones_v, cnt_sh) = rest
        else:
            (out_hbm,
             si0, si1, di0, di1, buf0, buf1, acc_sh,
             ss0, ss1, ds0, ds1, gs0, gs1) = rest
        c = lax.axis_index("c")
        s = lax.axis_index("s")
        w = c * NS + s
        row0 = s * rows_per_sub
        start = w * CPW + jnp.minimum(w, EXTRA)
        n_i = CPW + jnp.where(w < EXTRA, 1, 0)

        sidx = (si0, si1)
        didx = (di0, di1)
        bufs = (buf0, buf1)
        ssem = (ss0, ss1)
        dsem = (ds0, ds1)
        gsem = (gs0, gs1)

        pltpu.async_copy(src_hbm.at[start], si0, ss0)
        pltpu.async_copy(dst_hbm.at[start], di0, ds0)
        pltpu.async_copy(src_hbm.at[start + 1], si1, ss1)
        pltpu.async_copy(dst_hbm.at[start + 1], di1, ds1)
        pltpu.sync_copy(zeros_hbm, acc_sh.at[pl.ds(row0, rows_per_sub)])
        if with_counts:
            pltpu.sync_copy(ones_hbm.at[pl.ds(0, CHUNK)], ones_v)
            pltpu.sync_copy(ones_hbm.at[pl.ds(CHUNK, rows_per_sub)],
                            cnt_sh.at[pl.ds(row0, rows_per_sub)])
        plsc.subcore_barrier()
        pltpu.make_async_copy(src_hbm.at[start], si0, ss0).wait()
        pltpu.async_copy(table_hbm.at[si0], buf0, gs0)

        def step(k, b):
            b1 = 1 - b
            kk = start + k

            @pl.when(k + 1 < n_i)
            def _():
                pltpu.make_async_copy(
                    src_hbm.at[kk + 1], sidx[b1], ssem[b1]).wait()
                pltpu.async_copy(table_hbm.at[sidx[b1]], bufs[b1], gsem[b1])

            @pl.when(k < n_i)
            def _():
                pltpu.make_async_copy(
                    table_hbm.at[sidx[b]], bufs[b], gsem[b]).wait()
                pltpu.make_async_copy(dst_hbm.at[kk], didx[b], dsem[b]).wait()
                pltpu.sync_copy(bufs[b], acc_sh.at[didx[b]], add=True)
                if with_counts:
                    pltpu.sync_copy(ones_v, cnt_sh.at[didx[b]], add=True)

                @pl.when(k + 2 < n_i)
                def _():
                    pltpu.async_copy(src_hbm.at[kk + 2], sidx[b], ssem[b])
                    pltpu.async_copy(dst_hbm.at[kk + 2], didx[b], dsem[b])

        def body(g, _):
            step(2 * g, 0)
            step(2 * g + 1, 1)
            return 0

        lax.fori_loop(0, CPW_MAX // 2 + 1, body, 0)
        plsc.subcore_barrier()
        pltpu.sync_copy(acc_sh.at[pl.ds(row0, rows_per_sub)],
                        out_hbm.at[c, pl.ds(row0, rows_per_sub)])
        if with_counts:
            pltpu.sync_copy(cnt_sh.at[pl.ds(row0, rows_per_sub)],
                            cnt_hbm.at[c, pl.ds(row0, rows_per_sub)])

    return seg_sum


_seg_sum_x = _seg_sum_sc(F_DIM, True)
_seg_sum_h = _seg_sum_sc(H_DIM, False)


def _hp(a, b):
    return jnp.dot(a, b, preferred_element_type=jnp.float32,
                   precision=lax.Precision.DEFAULT)


def _enc_body(p_ref, c_ref, x_ref, w_ref, b_ref, m_ref, h_ref, ic_ref):
    inv = 1.0 / (c_ref[0] + c_ref[1] + 1.0)
    m = (p_ref[0] + p_ref[1] + x_ref[...]) * inv
    m_ref[...] = m
    h_ref[...] = jnp.maximum(_hp(m, w_ref[...]) + b_ref[...], 0.0)
    ic_ref[...] = inv


def _deca_body(h_ref, x_ref, wx1_ref, bx1_ref, wx2_ref, bx2_ref, attr_ref):
    x_hat = _hp(jnp.maximum(_hp(h_ref[...], wx1_ref[...]) + bx1_ref[...],
                            0.0), wx2_ref[...]) + bx2_ref[...]
    attr_ref[...] = jnp.sqrt(
        jnp.sum((x_hat - x_ref[...]) ** 2, axis=1, keepdims=True) + 1e-12)


def _decb_body(s2_ref, h_ref, x_ref, m_ref, ic_ref, attr_ref,
               wh1_ref, bh1_ref, wh2_ref, bh2_ref,
               score_ref, attr_o_ref, neigh_ref):
    h_bar = (s2_ref[0] + s2_ref[1] + h_ref[...]) * ic_ref[...]
    m_hat = _hp(jnp.maximum(_hp(h_bar, wh1_ref[...]) + bh1_ref[...], 0.0),
                wh2_ref[...]) + bh2_ref[...]
    x = x_ref[...]
    attr = attr_ref[...]
    neigh = jnp.sqrt(jnp.sum((m_hat - m_ref[...]) ** 2, axis=1,
                             keepdims=True) + 1e-12)
    homo = jnp.sqrt(jnp.sum((m_hat - x) ** 2, axis=1, keepdims=True) + 1e-12)
    score_ref[...] = attr + neigh + 0.5 * homo
    attr_o_ref[...] = attr
    neigh_ref[...] = neigh


_ROWS = 2000


def kernel(x, edge_index, edge_weight, W_enc, b_enc,
           Wx1, bx1, Wx2, bx2, Wh1, bh1, Wh2, bh2):
    del edge_weight
    pad = jnp.zeros((2, CHUNK), jnp.int32)
    ei = jnp.concatenate([edge_index, pad], axis=1)
    src = ei[0].reshape(NCHUNKS + 1, CHUNK)
    dst = ei[1].reshape(NCHUNKS + 1, CHUNK)

    ones0 = jnp.concatenate(
        [jnp.ones((CHUNK, 16), jnp.float32),
         jnp.zeros((NPAD // NS, 16), jnp.float32)], axis=0)
    p1, cnts = _seg_sum_x(x, src, dst,
                          jnp.zeros((NPAD // NS, F_DIM), jnp.float32), ones0)
    cnt_col = cnts[:, :, 0:1]

    grid = N // _ROWS
    m, h, inv_cnt = pl.pallas_call(
        _enc_body,
        grid=(grid,),
        in_specs=[
            pl.BlockSpec((NC, _ROWS, F_DIM), lambda i: (0, i, 0)),
            pl.BlockSpec((NC, _ROWS, 1), lambda i: (0, i, 0)),
            pl.BlockSpec((_ROWS, F_DIM), lambda i: (i, 0)),
            pl.BlockSpec((F_DIM, H_DIM), lambda i: (0, 0)),
            pl.BlockSpec((1, H_DIM), lambda i: (0, 0)),
        ],
        out_specs=[
            pl.BlockSpec((_ROWS, F_DIM), lambda i: (i, 0)),
            pl.BlockSpec((_ROWS, H_DIM), lambda i: (i, 0)),
            pl.BlockSpec((_ROWS, 1), lambda i: (i, 0)),
        ],
        out_shape=[
            jax.ShapeDtypeStruct((N, F_DIM), jnp.float32),
            jax.ShapeDtypeStruct((N, H_DIM), jnp.float32),
            jax.ShapeDtypeStruct((N, 1), jnp.float32),
        ],
    )(p1, cnt_col, x, W_enc, b_enc.reshape(1, H_DIM))

    [p2] = _seg_sum_h(h, src, dst,
                      jnp.zeros((NPAD // NS, H_DIM), jnp.float32))

    attr1 = pl.pallas_call(
        _deca_body,
        grid=(grid,),
        in_specs=[
            pl.BlockSpec((_ROWS, H_DIM), lambda i: (i, 0)),
            pl.BlockSpec((_ROWS, F_DIM), lambda i: (i, 0)),
            pl.BlockSpec((H_DIM, DEC_H), lambda i: (0, 0)),
            pl.BlockSpec((1, DEC_H), lambda i: (0, 0)),
            pl.BlockSpec((DEC_H, F_DIM), lambda i: (0, 0)),
            pl.BlockSpec((1, F_DIM), lambda i: (0, 0)),
        ],
        out_specs=pl.BlockSpec((_ROWS, 1), lambda i: (i, 0)),
        out_shape=jax.ShapeDtypeStruct((N, 1), jnp.float32),
    )(h, x, Wx1, bx1.reshape(1, DEC_H), Wx2, bx2.reshape(1, F_DIM))

    score, attr_err, neigh_err = pl.pallas_call(
        _decb_body,
        grid=(grid,),
        in_specs=[
            pl.BlockSpec((NC, _ROWS, H_DIM), lambda i: (0, i, 0)),
            pl.BlockSpec((_ROWS, H_DIM), lambda i: (i, 0)),
            pl.BlockSpec((_ROWS, F_DIM), lambda i: (i, 0)),
            pl.BlockSpec((_ROWS, F_DIM), lambda i: (i, 0)),
            pl.BlockSpec((_ROWS, 1), lambda i: (i, 0)),
            pl.BlockSpec((_ROWS, 1), lambda i: (i, 0)),
            pl.BlockSpec((H_DIM, DEC_H), lambda i: (0, 0)),
            pl.BlockSpec((1, DEC_H), lambda i: (0, 0)),
            pl.BlockSpec((DEC_H, F_DIM), lambda i: (0, 0)),
            pl.BlockSpec((1, F_DIM), lambda i: (0, 0)),
        ],
        out_specs=[
            pl.BlockSpec((_ROWS, 1), lambda i: (i, 0)),
            pl.BlockSpec((_ROWS, 1), lambda i: (i, 0)),
            pl.BlockSpec((_ROWS, 1), lambda i: (i, 0)),
        ],
        out_shape=[
            jax.ShapeDtypeStruct((N, 1), jnp.float32),
            jax.ShapeDtypeStruct((N, 1), jnp.float32),
            jax.ShapeDtypeStruct((N, 1), jnp.float32),
        ],
    )(p2, h, x, m, inv_cnt, attr1,
      Wh1, bh1.reshape(1, DEC_H), Wh2, bh2.reshape(1, F_DIM))

    return (score[:, 0], attr_err[:, 0], neigh_err[:, 0])


# speedup vs baseline: 18.6131x; 1.0444x over previous
import functools

import jax
import jax.numpy as jnp
from jax import lax
from jax.experimental import pallas as pl
from jax.experimental.pallas import tpu as pltpu
from jax.experimental.pallas import tpu_sc as plsc

N = 10000
NPAD = 10000
E = 320000
F_DIM = 128
H_DIM = 64
DEC_H = 128

NC = 2
NS = 16
NW = NC * NS
CHUNK = 128
NCHUNKS = E // CHUNK
CPW = NCHUNKS // NW
CPW_MAX = CPW + 1
EXTRA = NCHUNKS - CPW * NW


def _seg_sum_sc(d, with_counts):
    rows_per_sub = NPAD // NS

    mesh = plsc.VectorSubcoreMesh(
        core_axis_name="c", subcore_axis_name="s",
        num_cores=NC, num_subcores=NS)

    out_type = [jax.ShapeDtypeStruct((NC, NPAD, d), jnp.float32)]
    scratch = [
        pltpu.VMEM((CHUNK,), jnp.int32),
        pltpu.VMEM((CHUNK,), jnp.int32),
        pltpu.VMEM((CHUNK,), jnp.int32),
        pltpu.VMEM((CHUNK,), jnp.int32),
        pltpu.VMEM((CHUNK, d), jnp.float32),
        pltpu.VMEM((CHUNK, d), jnp.float32),
        pltpu.VMEM_SHARED((NPAD, d), jnp.float32),
        pltpu.SemaphoreType.DMA,
        pltpu.SemaphoreType.DMA,
        pltpu.SemaphoreType.DMA,
        pltpu.SemaphoreType.DMA,
        pltpu.SemaphoreType.DMA,
        pltpu.SemaphoreType.DMA,
    ]
    if with_counts:
        out_type.append(jax.ShapeDtypeStruct((NC, NPAD, 16), jnp.float32))
        scratch += [
            pltpu.VMEM((CHUNK, 16), jnp.float32),
            pltpu.VMEM_SHARED((NPAD, 16), jnp.float32),
        ]

    @functools.partial(
        pl.kernel,
        out_type=out_type,
        mesh=mesh,
        scratch_types=scratch,
        compiler_params=pltpu.CompilerParams(use_tc_tiling_on_sc=False),
    )
    def seg_sum(table_hbm, src_hbm, dst_hbm, zeros_hbm, *rest):
        if with_counts:
            (ones_hbm, out_hbm, cnt_hbm,
             si0, si1, di0, di1, buf0, buf1, acc_sh,
             ss0, ss1, ds0, ds1, gs0, gs1, ones_v, cnt_sh) = rest
        else:
            (out_hbm,
             si0, si1, di0, di1, buf0, buf1, acc_sh,
             ss0, ss1, ds0, ds1, gs0, gs1) = rest
        c = lax.axis_index("c")
        s = lax.axis_index("s")
        w = c * NS + s
        row0 = s * rows_per_sub
        start = w * CPW + jnp.minimum(w, EXTRA)
        n_i = CPW + jnp.where(w < EXTRA, 1, 0)

        sidx = (si0, si1)
        didx = (di0, di1)
        bufs = (buf0, buf1)
        ssem = (ss0, ss1)
        dsem = (ds0, ds1)
        gsem = (gs0, gs1)

        pltpu.async_copy(src_hbm.at[start], si0, ss0)
        pltpu.async_copy(dst_hbm.at[start], di0, ds0)
        pltpu.async_copy(src_hbm.at[start + 1], si1, ss1)
        pltpu.async_copy(dst_hbm.at[start + 1], di1, ds1)
        pltpu.sync_copy(zeros_hbm, acc_sh.at[pl.ds(row0, rows_per_sub)])
        if with_counts:
            pltpu.sync_copy(ones_hbm.at[pl.ds(0, CHUNK)], ones_v)
            pltpu.sync_copy(ones_hbm.at[pl.ds(CHUNK, rows_per_sub)],
                            cnt_sh.at[pl.ds(row0, rows_per_sub)])
        plsc.subcore_barrier()
        pltpu.make_async_copy(src_hbm.at[start], si0, ss0).wait()
        pltpu.async_copy(table_hbm.at[si0], buf0, gs0)

        def step(k, b):
            b1 = 1 - b
            kk = start + k

            @pl.when(k + 1 < n_i)
            def _():
                pltpu.make_async_copy(
                    src_hbm.at[kk + 1], sidx[b1], ssem[b1]).wait()
                pltpu.async_copy(table_hbm.at[sidx[b1]], bufs[b1], gsem[b1])

            @pl.when(k < n_i)
            def _():
                pltpu.make_async_copy(
                    table_hbm.at[sidx[b]], bufs[b], gsem[b]).wait()
                pltpu.make_async_copy(dst_hbm.at[kk], didx[b], dsem[b]).wait()
                pltpu.sync_copy(bufs[b], acc_sh.at[didx[b]], add=True)
                if with_counts:
                    pltpu.sync_copy(ones_v, cnt_sh.at[didx[b]], add=True)

                @pl.when(k + 2 < n_i)
                def _():
                    pltpu.async_copy(src_hbm.at[kk + 2], sidx[b], ssem[b])
                    pltpu.async_copy(dst_hbm.at[kk + 2], didx[b], dsem[b])

        def body(g, _):
            step(2 * g, 0)
            step(2 * g + 1, 1)
            return 0

        lax.fori_loop(0, CPW_MAX // 2 + 1, body, 0)
        plsc.subcore_barrier()
        pltpu.sync_copy(acc_sh.at[pl.ds(row0, rows_per_sub)],
                        out_hbm.at[c, pl.ds(row0, rows_per_sub)])
        if with_counts:
            pltpu.sync_copy(cnt_sh.at[pl.ds(row0, rows_per_sub)],
                            cnt_hbm.at[c, pl.ds(row0, rows_per_sub)])

    return seg_sum


_seg_sum_x = _seg_sum_sc(F_DIM, True)
_seg_sum_h = _seg_sum_sc(H_DIM, False)


def _hp(a, b):
    return jnp.dot(a, b, preferred_element_type=jnp.float32,
                   precision=lax.Precision.DEFAULT)


def _rowsum_t(v):
    return lax.dot_general(jnp.ones((1, v.shape[1]), jnp.float32), v,
                           dimension_numbers=(((1,), (1,)), ((), ())),
                           preferred_element_type=jnp.float32,
                           precision=lax.Precision.HIGHEST)


def _enc_body(p_ref, c_ref, x_ref, w_ref, b_ref, m_ref, h_ref, ic_ref):
    inv = 1.0 / (c_ref[0] + c_ref[1] + 1.0)
    m = (p_ref[0] + p_ref[1] + x_ref[...]) * inv
    m_ref[...] = m
    h_ref[...] = jnp.maximum(_hp(m, w_ref[...]) + b_ref[...], 0.0)
    ic_ref[...] = inv


def _deca_body(h_ref, x_ref, wx1_ref, bx1_ref, wx2_ref, bx2_ref, attr_ref):
    x_hat = _hp(jnp.maximum(_hp(h_ref[...], wx1_ref[...]) + bx1_ref[...],
                            0.0), wx2_ref[...]) + bx2_ref[...]
    d = x_hat - x_ref[...]
    attr_ref[pl.ds(pl.program_id(0), 1), :] = jnp.sqrt(
        _rowsum_t(d * d) + 1e-12)


def _decb_body(s2_ref, h_ref, x_ref, m_ref, ic_ref, attr_ref,
               wh1_ref, bh1_ref, wh2_ref, bh2_ref,
               score_ref, attr_o_ref, neigh_ref):
    h_bar = (s2_ref[0] + s2_ref[1] + h_ref[...]) * ic_ref[...]
    m_hat = _hp(jnp.maximum(_hp(h_bar, wh1_ref[...]) + bh1_ref[...], 0.0),
                wh2_ref[...]) + bh2_ref[...]
    x = x_ref[...]
    i = pl.ds(pl.program_id(0), 1)
    attr = attr_ref[i, :]
    dn = m_hat - m_ref[...]
    dh = m_hat - x
    neigh = jnp.sqrt(_rowsum_t(dn * dn) + 1e-12)
    homo = jnp.sqrt(_rowsum_t(dh * dh) + 1e-12)
    score_ref[i, :] = attr + neigh + 0.5 * homo
    attr_o_ref[i, :] = attr
    neigh_ref[i, :] = neigh


_ROWS = 2000


def kernel(x, edge_index, edge_weight, W_enc, b_enc,
           Wx1, bx1, Wx2, bx2, Wh1, bh1, Wh2, bh2):
    del edge_weight
    pad = jnp.zeros((2, CHUNK), jnp.int32)
    ei = jnp.concatenate([edge_index, pad], axis=1)
    src = ei[0].reshape(NCHUNKS + 1, CHUNK)
    dst = ei[1].reshape(NCHUNKS + 1, CHUNK)

    ones0 = jnp.concatenate(
        [jnp.ones((CHUNK, 16), jnp.float32),
         jnp.zeros((NPAD // NS, 16), jnp.float32)], axis=0)
    p1, cnts = _seg_sum_x(x, src, dst,
                          jnp.zeros((NPAD // NS, F_DIM), jnp.float32), ones0)
    cnt_col = cnts[:, :, 0:1]

    grid = N // _ROWS
    m, h, inv_cnt = pl.pallas_call(
        _enc_body,
        grid=(grid,),
        in_specs=[
            pl.BlockSpec((NC, _ROWS, F_DIM), lambda i: (0, i, 0)),
            pl.BlockSpec((NC, _ROWS, 1), lambda i: (0, i, 0)),
            pl.BlockSpec((_ROWS, F_DIM), lambda i: (i, 0)),
            pl.BlockSpec((F_DIM, H_DIM), lambda i: (0, 0)),
            pl.BlockSpec((1, H_DIM), lambda i: (0, 0)),
        ],
        out_specs=[
            pl.BlockSpec((_ROWS, F_DIM), lambda i: (i, 0)),
            pl.BlockSpec((_ROWS, H_DIM), lambda i: (i, 0)),
            pl.BlockSpec((_ROWS, 1), lambda i: (i, 0)),
        ],
        out_shape=[
            jax.ShapeDtypeStruct((N, F_DIM), jnp.float32),
            jax.ShapeDtypeStruct((N, H_DIM), jnp.float32),
            jax.ShapeDtypeStruct((N, 1), jnp.float32),
        ],
    )(p1, cnt_col, x, W_enc, b_enc.reshape(1, H_DIM))

    [p2] = _seg_sum_h(h, src, dst,
                      jnp.zeros((NPAD // NS, H_DIM), jnp.float32))

    attr1 = pl.pallas_call(
        _deca_body,
        grid=(grid,),
        in_specs=[
            pl.BlockSpec((_ROWS, H_DIM), lambda i: (i, 0)),
            pl.BlockSpec((_ROWS, F_DIM), lambda i: (i, 0)),
            pl.BlockSpec((H_DIM, DEC_H), lambda i: (0, 0)),
            pl.BlockSpec((1, DEC_H), lambda i: (0, 0)),
            pl.BlockSpec((DEC_H, F_DIM), lambda i: (0, 0)),
            pl.BlockSpec((1, F_DIM), lambda i: (0, 0)),
        ],
        out_specs=pl.BlockSpec((N // _ROWS, _ROWS), lambda i: (0, 0)),
        out_shape=jax.ShapeDtypeStruct((N // _ROWS, _ROWS), jnp.float32),
    )(h, x, Wx1, bx1.reshape(1, DEC_H), Wx2, bx2.reshape(1, F_DIM))

    score, attr_err, neigh_err = pl.pallas_call(
        _decb_body,
        grid=(grid,),
        in_specs=[
            pl.BlockSpec((NC, _ROWS, H_DIM), lambda i: (0, i, 0)),
            pl.BlockSpec((_ROWS, H_DIM), lambda i: (i, 0)),
            pl.BlockSpec((_ROWS, F_DIM), lambda i: (i, 0)),
            pl.BlockSpec((_ROWS, F_DIM), lambda i: (i, 0)),
            pl.BlockSpec((_ROWS, 1), lambda i: (i, 0)),
            pl.BlockSpec((N // _ROWS, _ROWS), lambda i: (0, 0)),
            pl.BlockSpec((H_DIM, DEC_H), lambda i: (0, 0)),
            pl.BlockSpec((1, DEC_H), lambda i: (0, 0)),
            pl.BlockSpec((DEC_H, F_DIM), lambda i: (0, 0)),
            pl.BlockSpec((1, F_DIM), lambda i: (0, 0)),
        ],
        out_specs=[
            pl.BlockSpec((N // _ROWS, _ROWS), lambda i: (0, 0)),
            pl.BlockSpec((N // _ROWS, _ROWS), lambda i: (0, 0)),
            pl.BlockSpec((N // _ROWS, _ROWS), lambda i: (0, 0)),
        ],
        out_shape=[
            jax.ShapeDtypeStruct((N // _ROWS, _ROWS), jnp.float32),
            jax.ShapeDtypeStruct((N // _ROWS, _ROWS), jnp.float32),
            jax.ShapeDtypeStruct((N // _ROWS, _ROWS), jnp.float32),
        ],
    )(p2, h, x, m, inv_cnt, attr1,
      Wh1, bh1.reshape(1, DEC_H), Wh2, bh2.reshape(1, F_DIM))

    return (score.reshape(N), attr_err.reshape(N), neigh_err.reshape(N))


# speedup vs baseline: 18.6401x; 1.0015x over previous
import functools

import jax
import jax.numpy as jnp
from jax import lax
from jax.experimental import pallas as pl
from jax.experimental.pallas import tpu as pltpu
from jax.experimental.pallas import tpu_sc as plsc

N = 10000
NPAD = 10000
E = 320000
F_DIM = 128
H_DIM = 64
DEC_H = 128

NC = 2
NS = 16
NW = NC * NS
CHUNK = 128
NCHUNKS = E // CHUNK
CPW = NCHUNKS // NW
CPW_MAX = CPW + 1
EXTRA = NCHUNKS - CPW * NW


def _seg_sum_sc(d, with_counts):
    rows_per_sub = NPAD // NS

    mesh = plsc.VectorSubcoreMesh(
        core_axis_name="c", subcore_axis_name="s",
        num_cores=NC, num_subcores=NS)

    out_type = [jax.ShapeDtypeStruct((NC, NPAD, d), jnp.float32)]
    scratch = [
        pltpu.VMEM((CHUNK,), jnp.int32),
        pltpu.VMEM((CHUNK,), jnp.int32),
        pltpu.VMEM((CHUNK,), jnp.int32),
        pltpu.VMEM((CHUNK,), jnp.int32),
        pltpu.VMEM((CHUNK, d), jnp.float32),
        pltpu.VMEM((CHUNK, d), jnp.float32),
        pltpu.VMEM_SHARED((NPAD, d), jnp.float32),
        pltpu.SemaphoreType.DMA,
        pltpu.SemaphoreType.DMA,
        pltpu.SemaphoreType.DMA,
        pltpu.SemaphoreType.DMA,
        pltpu.SemaphoreType.DMA,
        pltpu.SemaphoreType.DMA,
    ]
    if with_counts:
        out_type.append(jax.ShapeDtypeStruct((NC, NPAD, 16), jnp.float32))
        scratch += [
            pltpu.VMEM((CHUNK, 16), jnp.float32),
            pltpu.VMEM_SHARED((NPAD, 16), jnp.float32),
        ]

    @functools.partial(
        pl.kernel,
        out_type=out_type,
        mesh=mesh,
        scratch_types=scratch,
        compiler_params=pltpu.CompilerParams(use_tc_tiling_on_sc=False),
    )
    def seg_sum(table_hbm, src_hbm, dst_hbm, zeros_hbm, *rest):
        if with_counts:
            (ones_hbm, out_hbm, cnt_hbm,
             si0, si1, di0, di1, buf0, buf1, acc_sh,
             ss0, ss1, ds0, ds1, gs0, gs1, ones_v, cnt_sh) = rest
        else:
            (out_hbm,
             si0, si1, di0, di1, buf0, buf1, acc_sh,
             ss0, ss1, ds0, ds1, gs0, gs1) = rest
        c = lax.axis_index("c")
        s = lax.axis_index("s")
        w = c * NS + s
        row0 = s * rows_per_sub
        start = w * CPW + jnp.minimum(w, EXTRA)
        n_i = CPW + jnp.where(w < EXTRA, 1, 0)

        sidx = (si0, si1)
        didx = (di0, di1)
        bufs = (buf0, buf1)
        ssem = (ss0, ss1)
        dsem = (ds0, ds1)
        gsem = (gs0, gs1)

        pltpu.async_copy(src_hbm.at[start], si0, ss0)
        pltpu.async_copy(dst_hbm.at[start], di0, ds0)
        pltpu.async_copy(src_hbm.at[start + 1], si1, ss1)
        pltpu.async_copy(dst_hbm.at[start + 1], di1, ds1)
        pltpu.sync_copy(zeros_hbm, acc_sh.at[pl.ds(row0, rows_per_sub)])
        if with_counts:
            pltpu.sync_copy(ones_hbm.at[pl.ds(0, CHUNK)], ones_v)
            pltpu.sync_copy(ones_hbm.at[pl.ds(CHUNK, rows_per_sub)],
                            cnt_sh.at[pl.ds(row0, rows_per_sub)])
        plsc.subcore_barrier()
        pltpu.make_async_copy(src_hbm.at[start], si0, ss0).wait()
        pltpu.async_copy(table_hbm.at[si0], buf0, gs0)

        def step(k, b):
            b1 = 1 - b
            kk = start + k

            @pl.when(k + 1 < n_i)
            def _():
                pltpu.make_async_copy(
                    src_hbm.at[kk + 1], sidx[b1], ssem[b1]).wait()
                pltpu.async_copy(table_hbm.at[sidx[b1]], bufs[b1], gsem[b1])

            @pl.when(k < n_i)
            def _():
                pltpu.make_async_copy(
                    table_hbm.at[sidx[b]], bufs[b], gsem[b]).wait()
                pltpu.make_async_copy(dst_hbm.at[kk], didx[b], dsem[b]).wait()
                pltpu.sync_copy(bufs[b], acc_sh.at[didx[b]], add=True)
                if with_counts:
                    pltpu.sync_copy(ones_v, cnt_sh.at[didx[b]], add=True)

                @pl.when(k + 2 < n_i)
                def _():
                    pltpu.async_copy(src_hbm.at[kk + 2], sidx[b], ssem[b])
                    pltpu.async_copy(dst_hbm.at[kk + 2], didx[b], dsem[b])

        def body(g, _):
            step(2 * g, 0)
            step(2 * g + 1, 1)
            return 0

        lax.fori_loop(0, CPW_MAX // 2 + 1, body, 0)
        plsc.subcore_barrier()
        pltpu.sync_copy(acc_sh.at[pl.ds(row0, rows_per_sub)],
                        out_hbm.at[c, pl.ds(row0, rows_per_sub)])
        if with_counts:
            pltpu.sync_copy(cnt_sh.at[pl.ds(row0, rows_per_sub)],
                            cnt_hbm.at[c, pl.ds(row0, rows_per_sub)])

    return seg_sum


_seg_sum_x = _seg_sum_sc(F_DIM, True)
_seg_sum_h = _seg_sum_sc(H_DIM, False)


def _hp(a, b):
    return jnp.dot(a, b, preferred_element_type=jnp.float32,
                   precision=lax.Precision.DEFAULT)


def _rowsum_t(v):
    return lax.dot_general(jnp.ones((1, v.shape[1]), jnp.float32), v,
                           dimension_numbers=(((1,), (1,)), ((), ())),
                           preferred_element_type=jnp.float32,
                           precision=lax.Precision.HIGHEST)


def _enc_body(p_ref, c_ref, x_ref, w_ref, b_ref, m_ref, h_ref, ic_ref):
    inv = 1.0 / (c_ref[0] + c_ref[1] + 1.0)
    m = (p_ref[0] + p_ref[1] + x_ref[...]) * inv
    m_ref[...] = m
    h_ref[...] = jnp.maximum(_hp(m, w_ref[...]) + b_ref[...], 0.0)
    ic_ref[...] = inv


def _deca_body(h_ref, x_ref, wx1_ref, bx1_ref, wx2_ref, bx2_ref, attr_ref):
    x_hat = _hp(jnp.maximum(_hp(h_ref[...], wx1_ref[...]) + bx1_ref[...],
                            0.0), wx2_ref[...]) + bx2_ref[...]
    d = x_hat - x_ref[...]
    attr_ref[pl.ds(pl.program_id(0), 1), :] = jnp.sqrt(
        _rowsum_t(d * d) + 1e-12)


def _decb_body(s2_ref, h_ref, x_ref, m_ref, ic_ref, attr_ref,
               wh1_ref, bh1_ref, wh2_ref, bh2_ref,
               score_ref, attr_o_ref, neigh_ref):
    h_bar = (s2_ref[0] + s2_ref[1] + h_ref[...]) * ic_ref[...]
    m_hat = _hp(jnp.maximum(_hp(h_bar, wh1_ref[...]) + bh1_ref[...], 0.0),
                wh2_ref[...]) + bh2_ref[...]
    x = x_ref[...]
    i = pl.ds(pl.program_id(0), 1)
    attr = attr_ref[i, :]
    dn = m_hat - m_ref[...]
    dh = m_hat - x
    neigh = jnp.sqrt(_rowsum_t(dn * dn) + 1e-12)
    homo = jnp.sqrt(_rowsum_t(dh * dh) + 1e-12)
    score_ref[i, :] = attr + neigh + 0.5 * homo
    attr_o_ref[i, :] = attr
    neigh_ref[i, :] = neigh


_ROWS = 2000


def kernel(x, edge_index, edge_weight, W_enc, b_enc,
           Wx1, bx1, Wx2, bx2, Wh1, bh1, Wh2, bh2):
    del edge_weight
    src = edge_index[0].reshape(NCHUNKS, CHUNK)
    dst = edge_index[1].reshape(NCHUNKS, CHUNK)

    ones0 = jnp.concatenate(
        [jnp.ones((CHUNK, 16), jnp.float32),
         jnp.zeros((NPAD // NS, 16), jnp.float32)], axis=0)
    p1, cnts = _seg_sum_x(x, src, dst,
                          jnp.zeros((NPAD // NS, F_DIM), jnp.float32), ones0)
    cnt_col = cnts[:, :, 0:1]

    grid = N // _ROWS
    m, h, inv_cnt = pl.pallas_call(
        _enc_body,
        grid=(grid,),
        in_specs=[
            pl.BlockSpec((NC, _ROWS, F_DIM), lambda i: (0, i, 0)),
            pl.BlockSpec((NC, _ROWS, 1), lambda i: (0, i, 0)),
            pl.BlockSpec((_ROWS, F_DIM), lambda i: (i, 0)),
            pl.BlockSpec((F_DIM, H_DIM), lambda i: (0, 0)),
            pl.BlockSpec((1, H_DIM), lambda i: (0, 0)),
        ],
        out_specs=[
            pl.BlockSpec((_ROWS, F_DIM), lambda i: (i, 0)),
            pl.BlockSpec((_ROWS, H_DIM), lambda i: (i, 0)),
            pl.BlockSpec((_ROWS, 1), lambda i: (i, 0)),
        ],
        out_shape=[
            jax.ShapeDtypeStruct((N, F_DIM), jnp.float32),
            jax.ShapeDtypeStruct((N, H_DIM), jnp.float32),
            jax.ShapeDtypeStruct((N, 1), jnp.float32),
        ],
    )(p1, cnt_col, x, W_enc, b_enc.reshape(1, H_DIM))

    [p2] = _seg_sum_h(h, src, dst,
                      jnp.zeros((NPAD // NS, H_DIM), jnp.float32))

    attr1 = pl.pallas_call(
        _deca_body,
        grid=(grid,),
        in_specs=[
            pl.BlockSpec((_ROWS, H_DIM), lambda i: (i, 0)),
            pl.BlockSpec((_ROWS, F_DIM), lambda i: (i, 0)),
            pl.BlockSpec((H_DIM, DEC_H), lambda i: (0, 0)),
            pl.BlockSpec((1, DEC_H), lambda i: (0, 0)),
            pl.BlockSpec((DEC_H, F_DIM), lambda i: (0, 0)),
            pl.BlockSpec((1, F_DIM), lambda i: (0, 0)),
        ],
        out_specs=pl.BlockSpec((N // _ROWS, _ROWS), lambda i: (0, 0)),
        out_shape=jax.ShapeDtypeStruct((N // _ROWS, _ROWS), jnp.float32),
    )(h, x, Wx1, bx1.reshape(1, DEC_H), Wx2, bx2.reshape(1, F_DIM))

    score, attr_err, neigh_err = pl.pallas_call(
        _decb_body,
        grid=(grid,),
        in_specs=[
            pl.BlockSpec((NC, _ROWS, H_DIM), lambda i: (0, i, 0)),
            pl.BlockSpec((_ROWS, H_DIM), lambda i: (i, 0)),
            pl.BlockSpec((_ROWS, F_DIM), lambda i: (i, 0)),
            pl.BlockSpec((_ROWS, F_DIM), lambda i: (i, 0)),
            pl.BlockSpec((_ROWS, 1), lambda i: (i, 0)),
            pl.BlockSpec((N // _ROWS, _ROWS), lambda i: (0, 0)),
            pl.BlockSpec((H_DIM, DEC_H), lambda i: (0, 0)),
            pl.BlockSpec((1, DEC_H), lambda i: (0, 0)),
            pl.BlockSpec((DEC_H, F_DIM), lambda i: (0, 0)),
            pl.BlockSpec((1, F_DIM), lambda i: (0, 0)),
        ],
        out_specs=[
            pl.BlockSpec((N // _ROWS, _ROWS), lambda i: (0, 0)),
            pl.BlockSpec((N // _ROWS, _ROWS), lambda i: (0, 0)),
            pl.BlockSpec((N // _ROWS, _ROWS), lambda i: (0, 0)),
        ],
        out_shape=[
            jax.ShapeDtypeStruct((N // _ROWS, _ROWS), jnp.float32),
            jax.ShapeDtypeStruct((N // _ROWS, _ROWS), jnp.float32),
            jax.ShapeDtypeStruct((N // _ROWS, _ROWS), jnp.float32),
        ],
    )(p2, h, x, m, inv_cnt, attr1,
      Wh1, bh1.reshape(1, DEC_H), Wh2, bh2.reshape(1, F_DIM))

    return (score.reshape(N), attr_err.reshape(N), neigh_err.reshape(N))
